# Initial kernel scaffold; baseline (speedup 1.0000x reference)
import jax
import jax.numpy as jnp
from jax.experimental import pallas as pl


def kernel(x_nchw, mask_nchw, conv_w_oihw, gamma, beta):
    raise NotImplementedError("write your pallas kernel here")


# speedup vs baseline: 5.4929x; 5.4929x over previous
import functools

import jax
import jax.numpy as jnp
from jax.experimental import pallas as pl
from jax.experimental.pallas import tpu as pltpu

_K = 3
_PAD = 1
_EPS = 1e-5
_TH = 2


@jax.jit
def _forward(x_nchw, mask_nchw, conv_w_oihw, gamma, beta):
    B, Cin, H, W = x_nchw.shape
    Cout = conv_w_oihw.shape[0]
    Hout, Wout = H, W
    Hp, Wp = H + 2 * _PAD, W + 2 * _PAD
    WPC = Wp * Cin
    CW = Cout * Wout
    NHT = Hout // _TH
    M = _TH * B

    xt = jnp.transpose(x_nchw, (2, 0, 3, 1)).astype(jnp.bfloat16)
    xt = jnp.pad(xt, ((_PAD, _PAD), (0, 0), (_PAD, _PAD), (0, 0)))
    x_slab = xt.reshape(Hp * B, WPC)

    wt = jnp.transpose(conv_w_oihw, (2, 3, 1, 0)).astype(jnp.float32)
    sel = jnp.stack([jnp.eye(Wout, Wp, k=kw, dtype=jnp.float32)
                     for kw in range(_K)])
    band = jnp.einsum('xwp,kxio->kpiow', sel, wt)
    w_band = band.reshape(_K, WPC, CW).astype(jnp.bfloat16)

    mt = jnp.transpose(mask_nchw.reshape(B, Hout, Wout), (1, 0, 2))
    m_slab = mt.reshape(Hout * B, Wout).astype(jnp.bfloat16)
    expand = jnp.tile(jnp.eye(Wout, dtype=jnp.bfloat16), (1, Cout))

    def conv_kernel(x_ref, w_ref, m_ref, e_ref, y_ref, st_ref):
        ht = pl.program_id(0)
        r0 = pl.multiple_of(ht * M, M)
        acc = jnp.zeros((M, CW), jnp.float32)
        for kh in range(_K):
            lhs = x_ref[pl.ds(r0 + kh * B, M), :]
            acc = acc + jnp.dot(lhs, w_ref[kh],
                                preferred_element_type=jnp.float32)
        mexp = jnp.dot(m_ref[pl.ds(r0, M), :], e_ref[...],
                       preferred_element_type=jnp.float32)
        ym = acc * mexp
        y_ref[...] = ym
        st_ref[0, 0:1, :] = jnp.sum(ym, axis=0, keepdims=True)
        st_ref[0, 1:2, :] = jnp.sum(ym * ym, axis=0, keepdims=True)

    y, st = pl.pallas_call(
        conv_kernel,
        out_shape=(jax.ShapeDtypeStruct((Hout * B, CW), jnp.float32),
                   jax.ShapeDtypeStruct((NHT, 2, CW), jnp.float32)),
        grid=(NHT,),
        in_specs=[
            pl.BlockSpec((Hp * B, WPC), lambda ht: (0, 0)),
            pl.BlockSpec((_K, WPC, CW), lambda ht: (0, 0, 0)),
            pl.BlockSpec((Hout * B, Wout), lambda ht: (0, 0)),
            pl.BlockSpec((Wout, CW), lambda ht: (0, 0)),
        ],
        out_specs=(
            pl.BlockSpec((M, CW), lambda ht: (ht, 0)),
            pl.BlockSpec((1, 2, CW), lambda ht: (ht, 0, 0)),
        ),
        compiler_params=pltpu.CompilerParams(
            dimension_semantics=("parallel",),
            vmem_limit_bytes=64 * 1024 * 1024),
    )(x_slab, w_band, m_slab, expand)

    tot = jnp.sum(st.reshape(NHT, 2, Cout, Wout), axis=(0, 3))
    n = float(B * Hout * Wout)
    mean = tot[0] / n
    var = jnp.maximum(tot[1] / n - mean * mean, 0.0)
    scale = gamma.astype(jnp.float32) / jnp.sqrt(var + _EPS)
    shift = beta.astype(jnp.float32) - mean * scale
    scale_row = jnp.repeat(scale, Wout).reshape(1, CW)
    shift_row = jnp.repeat(shift, Wout).reshape(1, CW)

    R = Hout * B
    TR = 512

    def bn_relu_kernel(y_ref, s_ref, t_ref, o_ref):
        o_ref[...] = jnp.maximum(y_ref[...] * s_ref[...] + t_ref[...], 0.0)

    z = pl.pallas_call(
        bn_relu_kernel,
        out_shape=jax.ShapeDtypeStruct((R, CW), jnp.float32),
        grid=(R // TR,),
        in_specs=[
            pl.BlockSpec((TR, CW), lambda i: (i, 0)),
            pl.BlockSpec((1, CW), lambda i: (0, 0)),
            pl.BlockSpec((1, CW), lambda i: (0, 0)),
        ],
        out_specs=pl.BlockSpec((TR, CW), lambda i: (i, 0)),
        compiler_params=pltpu.CompilerParams(
            dimension_semantics=("parallel",),
            vmem_limit_bytes=64 * 1024 * 1024),
    )(y, scale_row, shift_row)

    out = z.reshape(Hout, B, Cout, Wout).transpose(1, 2, 0, 3)
    return out, mask_nchw


def kernel(x_nchw, mask_nchw, conv_w_oihw, gamma, beta):
    if mask_nchw is None:
        ones = jnp.ones((x_nchw.shape[0], 1) + x_nchw.shape[2:], jnp.float32)
        out, _ = _forward(x_nchw, ones, conv_w_oihw, gamma, beta)
        return out, None
    return _forward(x_nchw, mask_nchw, conv_w_oihw, gamma, beta)


# speedup vs baseline: 5.8780x; 1.0701x over previous
import jax
import jax.numpy as jnp
from jax.experimental import pallas as pl
from jax.experimental.pallas import tpu as pltpu

_K = 3
_PAD = 1
_EPS = 1e-5
_TH = 2


@jax.jit
def _forward(x_nchw, mask_nchw, conv_w_oihw, gamma, beta):
    B, Cin, H, W = x_nchw.shape
    Cout = conv_w_oihw.shape[0]
    Hout, Wout = H, W
    Hp = H + 2 * _PAD
    WC = Wout * Cin
    CW = Cout * Wout
    NHT = Hout // _TH
    M = _TH * B
    n = float(B * Hout * Wout)

    xt = jnp.transpose(x_nchw, (2, 0, 3, 1)).astype(jnp.bfloat16)
    xt = jnp.pad(xt, ((_PAD, _PAD), (0, 0), (0, 0), (0, 0)))
    x_slab = xt.reshape(Hp * B, WC)

    wt = jnp.transpose(conv_w_oihw, (2, 3, 1, 0)).astype(jnp.float32)
    sel = jnp.stack([jnp.eye(Wout, Wout, k=kw - _PAD, dtype=jnp.float32)
                     for kw in range(_K)])
    band = jnp.einsum('xwp,kxio->kpiow', sel, wt)
    w_band = band.reshape(_K, WC, CW).astype(jnp.bfloat16)

    mt = jnp.transpose(mask_nchw.reshape(B, Hout, Wout), (1, 0, 2))
    m_slab = mt.reshape(Hout * B, Wout).astype(jnp.bfloat16)
    expand = jnp.tile(jnp.eye(Wout, dtype=jnp.bfloat16), (1, Cout))

    def conv_kernel(x_ref, w_ref, m_ref, e_ref, y_ref, st_ref):
        ht = pl.program_id(0)
        r0 = pl.multiple_of(ht * M, M)
        acc = jnp.zeros((M, CW), jnp.float32)
        for kh in range(_K):
            lhs = x_ref[pl.ds(r0 + kh * B, M), :]
            acc = acc + jnp.dot(lhs, w_ref[kh],
                                preferred_element_type=jnp.float32)
        mexp = jnp.dot(m_ref[pl.ds(r0, M), :], e_ref[...],
                       preferred_element_type=jnp.float32)
        ym = acc * mexp
        y_ref[...] = ym.astype(jnp.bfloat16)
        st_ref[0, 0:1, :] = jnp.sum(ym, axis=0, keepdims=True)
        st_ref[0, 1:2, :] = jnp.sum(ym * ym, axis=0, keepdims=True)

    y, st = pl.pallas_call(
        conv_kernel,
        out_shape=(jax.ShapeDtypeStruct((Hout * B, CW), jnp.bfloat16),
                   jax.ShapeDtypeStruct((NHT, 2, CW), jnp.float32)),
        grid=(NHT,),
        in_specs=[
            pl.BlockSpec((Hp * B, WC), lambda ht: (0, 0)),
            pl.BlockSpec((_K, WC, CW), lambda ht: (0, 0, 0)),
            pl.BlockSpec((Hout * B, Wout), lambda ht: (0, 0)),
            pl.BlockSpec((Wout, CW), lambda ht: (0, 0)),
        ],
        out_specs=(
            pl.BlockSpec((M, CW), lambda ht: (ht, 0)),
            pl.BlockSpec((1, 2, CW), lambda ht: (ht, 0, 0)),
        ),
        compiler_params=pltpu.CompilerParams(
            dimension_semantics=("parallel",),
            vmem_limit_bytes=64 * 1024 * 1024),
    )(x_slab, w_band, m_slab, expand)

    osum = jnp.kron(jnp.eye(Cout, dtype=jnp.float32),
                    jnp.ones((Wout, Wout), jnp.float32))
    rexp = jnp.repeat(jnp.eye(Cout, dtype=jnp.float32), Wout, axis=1)
    g_row = gamma.astype(jnp.float32).reshape(1, Cout)
    b_row = beta.astype(jnp.float32).reshape(1, Cout)

    R = Hout * B
    TR = min(1024, R)

    def bn_relu_kernel(y_ref, st_ref, o_ref, r_ref, g_ref, bt_ref, out_ref):
        s = jnp.sum(st_ref[...], axis=0)
        tot = jnp.dot(s, o_ref[...],
                      preferred_element_type=jnp.float32) * (1.0 / n)
        mean = tot[0:1, :]
        var = jnp.maximum(tot[1:2, :] - mean * mean, 0.0)
        gl = jnp.dot(g_ref[...], r_ref[...],
                     preferred_element_type=jnp.float32)
        bl = jnp.dot(bt_ref[...], r_ref[...],
                     preferred_element_type=jnp.float32)
        scale = gl / jnp.sqrt(var + _EPS)
        shift = bl - mean * scale
        yv = y_ref[...].astype(jnp.float32)
        out_ref[...] = jnp.maximum(yv * scale + shift, 0.0)

    z = pl.pallas_call(
        bn_relu_kernel,
        out_shape=jax.ShapeDtypeStruct((R, CW), jnp.float32),
        grid=(R // TR,),
        in_specs=[
            pl.BlockSpec((TR, CW), lambda i: (i, 0)),
            pl.BlockSpec((NHT, 2, CW), lambda i: (0, 0, 0)),
            pl.BlockSpec((CW, CW), lambda i: (0, 0)),
            pl.BlockSpec((Cout, CW), lambda i: (0, 0)),
            pl.BlockSpec((1, Cout), lambda i: (0, 0)),
            pl.BlockSpec((1, Cout), lambda i: (0, 0)),
        ],
        out_specs=pl.BlockSpec((TR, CW), lambda i: (i, 0)),
        compiler_params=pltpu.CompilerParams(
            dimension_semantics=("parallel",),
            vmem_limit_bytes=64 * 1024 * 1024),
    )(y, st, osum, rexp, g_row, b_row)

    out = z.reshape(Hout, B, Cout, Wout).transpose(1, 2, 0, 3)
    return out, mask_nchw


def kernel(x_nchw, mask_nchw, conv_w_oihw, gamma, beta):
    if mask_nchw is None:
        ones = jnp.ones((x_nchw.shape[0], 1) + x_nchw.shape[2:], jnp.float32)
        out, _ = _forward(x_nchw, ones, conv_w_oihw, gamma, beta)
        return out, None
    return _forward(x_nchw, mask_nchw, conv_w_oihw, gamma, beta)
